# baseline (speedup 1.0000x reference)
import functools

import numpy as np
import jax
import jax.numpy as jnp
from jax import lax
from jax.experimental import pallas as pl
from jax.experimental.pallas import tpu as pltpu
from jax.experimental.pallas import tpu_sc as plsc

_EMB = 64
_LANES = 16
_SCALE = float(np.sqrt(np.float32(_EMB)))


def _pos_encoding(length, depth):
    half = depth / 2
    positions = np.arange(length)[:, np.newaxis]
    depths = np.arange(half)[np.newaxis, :] / half
    angle_rates = 1 / 10000 ** depths
    angle_rads = positions * angle_rates
    return np.concatenate(
        [np.sin(angle_rads), np.cos(angle_rads)], axis=-1
    ).astype(np.float32)


@functools.partial(jax.jit, static_argnames=())
def _run(x, table, pos):
    B, L = x.shape
    info = plsc.get_sparse_core_info()
    NC, NS = info.num_cores, info.num_subcores
    NW = NC * NS
    SPW = B // NW
    C0 = 104
    C1 = L - C0

    mesh = plsc.VectorSubcoreMesh(core_axis_name="c", subcore_axis_name="s")

    @functools.partial(
        pl.kernel,
        out_type=jax.ShapeDtypeStruct((B, L, _EMB), jnp.float32),
        mesh=mesh,
        scratch_types=[
            pltpu.VMEM((L,), jnp.int32),
            pltpu.VMEM((L, _EMB), jnp.float32),
            pltpu.VMEM((L, _EMB), jnp.float32),
            pltpu.SemaphoreType.DMA,
        ],
        compiler_params=pltpu.CompilerParams(use_tc_tiling_on_sc=False),
    )
    def body(x_hbm, table_hbm, pos_hbm, out_hbm, idx_v, rows_v, pos_v, sem):
        wid = lax.axis_index("s") * NC + lax.axis_index("c")
        pltpu.sync_copy(pos_hbm, pos_v)

        def seq_body(i, carry):
            seq = wid * SPW + i
            pltpu.sync_copy(x_hbm.at[seq], idx_v)
            cp0 = pltpu.async_copy(
                table_hbm.at[idx_v.at[pl.ds(0, C0)]],
                rows_v.at[pl.ds(0, C0)], sem)
            cp1 = pltpu.async_copy(
                table_hbm.at[idx_v.at[pl.ds(C0, C1)]],
                rows_v.at[pl.ds(C0, C1)], sem)
            cp0.wait()
            cp1.wait()

            def row_body(r, c):
                for j in range(_EMB // _LANES):
                    sl = pl.ds(j * _LANES, _LANES)
                    rows_v[r, sl] = rows_v[r, sl] * _SCALE + pos_v[r, sl]
                return c

            lax.fori_loop(0, L, row_body, 0)
            pltpu.sync_copy(rows_v, out_hbm.at[seq])
            return carry

        lax.fori_loop(0, SPW, seq_body, 0)

    return body(x, table, pos)


def kernel(x, table):
    L = x.shape[1]
    pos = jnp.asarray(_pos_encoding(L, _EMB))
    return _run(x.astype(jnp.int32), table, pos)


# speedup vs baseline: 1.2142x; 1.2142x over previous
import functools

import numpy as np
import jax
import jax.numpy as jnp
from jax import lax
from jax.experimental import pallas as pl
from jax.experimental.pallas import tpu as pltpu
from jax.experimental.pallas import tpu_sc as plsc

_EMB = 64
_LANES = 16
_SCALE = float(np.sqrt(np.float32(_EMB)))
_NBUF = 4
_AHEAD = 2


def _pos_encoding(length, depth):
    half = depth / 2
    positions = np.arange(length)[:, np.newaxis]
    depths = np.arange(half)[np.newaxis, :] / half
    angle_rates = 1 / 10000 ** depths
    angle_rads = positions * angle_rates
    return np.concatenate(
        [np.sin(angle_rads), np.cos(angle_rads)], axis=-1
    ).astype(np.float32)


@jax.jit
def _run(x, table, pos):
    B, L = x.shape
    info = plsc.get_sparse_core_info()
    NC, NS = info.num_cores, info.num_subcores
    NW = NC * NS
    SPW = B // NW
    C0 = 104
    C1 = L - C0

    mesh = plsc.VectorSubcoreMesh(core_axis_name="c", subcore_axis_name="s")

    @functools.partial(
        pl.kernel,
        out_type=jax.ShapeDtypeStruct((B, L, _EMB), jnp.float32),
        mesh=mesh,
        scratch_types=[
            pltpu.VMEM((SPW, L), jnp.int32),
            pltpu.VMEM((L, _EMB), jnp.float32),
        ]
        + [pltpu.VMEM((L, _EMB), jnp.float32) for _ in range(_NBUF)]
        + [
            pltpu.SemaphoreType.DMA((_NBUF,)),
            pltpu.SemaphoreType.DMA((_NBUF,)),
        ],
        compiler_params=pltpu.CompilerParams(use_tc_tiling_on_sc=False),
    )
    def body(x_hbm, table_hbm, pos_hbm, out_hbm, idx_v, pos_v,
             r0, r1, r2, r3, gsem, osem):
        rows = [r0, r1, r2, r3]
        wid = lax.axis_index("s") * NC + lax.axis_index("c")
        base = wid * SPW
        pltpu.sync_copy(pos_hbm, pos_v)
        pltpu.sync_copy(x_hbm.at[pl.ds(base, SPW)], idx_v)

        def start_gather(seq_local, b):
            pltpu.async_copy(
                table_hbm.at[idx_v.at[seq_local, pl.ds(0, C0)]],
                rows[b].at[pl.ds(0, C0)], gsem.at[b])
            pltpu.async_copy(
                table_hbm.at[idx_v.at[seq_local, pl.ds(C0, C1)]],
                rows[b].at[pl.ds(C0, C1)], gsem.at[b])

        def wait_gather(b):
            pltpu.make_async_copy(
                table_hbm.at[idx_v.at[0, pl.ds(0, C0)]],
                rows[b].at[pl.ds(0, C0)], gsem.at[b]).wait()
            pltpu.make_async_copy(
                table_hbm.at[idx_v.at[0, pl.ds(C0, C1)]],
                rows[b].at[pl.ds(C0, C1)], gsem.at[b]).wait()

        def wait_out(seq_local, b):
            pltpu.make_async_copy(
                rows[b], out_hbm.at[base + seq_local], osem.at[b]).wait()

        for b in range(_AHEAD):
            start_gather(b, b)

        def group(k, carry):
            for b in range(_NBUF):
                i = k * _NBUF + b
                wait_gather(b)

                def row_body(r, c):
                    for rr in range(2):
                        for j in range(_EMB // _LANES):
                            sl = pl.ds(j * _LANES, _LANES)
                            rows[b][2 * r + rr, sl] = (
                                rows[b][2 * r + rr, sl] * _SCALE
                                + pos_v[2 * r + rr, sl])
                    return c

                lax.fori_loop(0, L // 2, row_body, 0, unroll=False)
                pltpu.async_copy(rows[b], out_hbm.at[base + i], osem.at[b])

                bg = (b + _AHEAD) % _NBUF

                @pl.when(i + _AHEAD < SPW)
                def _():
                    @pl.when(i + _AHEAD >= _NBUF)
                    def _():
                        wait_out(i + _AHEAD - _NBUF, bg)
                    start_gather(i + _AHEAD, bg)
            return carry

        lax.fori_loop(0, SPW // _NBUF, group, 0, unroll=False)
        for b in range(_NBUF):
            wait_out(SPW - _NBUF + b, b)

    return body(x, table, pos)


def kernel(x, table):
    L = x.shape[1]
    pos = jnp.asarray(_pos_encoding(L, _EMB))
    return _run(x.astype(jnp.int32), table, pos)
